# baseline (speedup 1.0000x reference)
import functools

import jax
import jax.numpy as jnp
from jax import lax
from jax.experimental import pallas as pl
from jax.experimental.pallas import tpu as pltpu
from jax.experimental.pallas import tpu_sc as plsc

B, L, E = 1024, 200, 128
N = B * L
POS_BASE = 256
CTAB = 2 * POS_BASE
PAD_ID = 0
EPS = 1e-12
GATHER_W = 128
BBA = 256
BB = 8
TOK = BB * L
NBLK = B // BB
CBLKS = (16, 24, 40, 48)


def _sc_gather_rows(table, idx_flat, off_blocks, cblk):
    mesh = plsc.VectorSubcoreMesh(core_axis_name="c", subcore_axis_name="s")
    nc = cblk * TOK
    base = off_blocks * TOK // GATHER_W

    @functools.partial(
        pl.kernel,
        out_type=jax.ShapeDtypeStruct((nc, E), jnp.float32),
        mesh=mesh,
    )
    def gather_kernel(x_hbm, i_hbm, o_hbm):
        def body(i_vmem, o_vmem):
            pltpu.sync_copy(x_hbm.at[i_vmem.at[0]], o_vmem)

        pltpu.emit_pipeline(
            body,
            grid=(nc // GATHER_W,),
            in_specs=[pl.BlockSpec((1, GATHER_W), lambda i: (0, i + base))],
            out_specs=[pl.BlockSpec((GATHER_W, E), lambda i: (i, 0))],
            core_axis_name=("c", "s"),
            dimension_semantics=(pltpu.PARALLEL,),
        )(i_hbm, o_hbm)

    return gather_kernel(table, idx_flat)


def _cidx_kernel(ids_ref, seg_ref, tri_ref, cidx_ref):
    ids = ids_ref[...]
    mask = ids != PAD_ID
    mbf = mask.astype(jnp.bfloat16)
    posf = lax.dot_general(mbf, tri_ref[...], (((1,), (0,)), ((), ())),
                           preferred_element_type=jnp.float32)
    pos = jnp.where(mask, posf.astype(jnp.int32), 0)
    cidx_ref[...] = pos + POS_BASE * seg_ref[...]


def _finish_body(w_ref, cidx_ref, tab_ref, gamma_ref, beta_ref, out_ref):
    cid_t = jnp.transpose(cidx_ref[0])
    oh = (cid_t == lax.broadcasted_iota(jnp.int32, (1, CTAB), 1))
    oh = oh.astype(jnp.bfloat16)
    d23 = lax.dot_general(oh, tab_ref[...], (((1,), (0,)), ((), ())),
                          preferred_element_type=jnp.float32)

    t = w_ref[0] + d23
    mean = jnp.mean(t, axis=1, keepdims=True)
    tcen = t - mean
    var = jnp.mean(tcen * tcen, axis=1, keepdims=True)
    y = tcen * lax.rsqrt(var + EPS) * gamma_ref[...] + beta_ref[...]
    out_ref[0] = y


def _finish_body_aliased(w_ref, cidx_ref, tab_ref, gamma_ref, beta_ref,
                         prev_ref, out_ref):
    del prev_ref
    _finish_body(w_ref, cidx_ref, tab_ref, gamma_ref, beta_ref, out_ref)


def kernel(input_ids, segment_ids, word_emb, seg_emb, pos_emb, ln_gamma,
           ln_beta):
    ids32 = input_ids.astype(jnp.int32)
    segs32 = segment_ids.astype(jnp.int32)

    ids_flat = ids32.reshape(1, N)
    offs = [sum(CBLKS[:c]) for c in range(len(CBLKS))]
    wrow_chunks = [_sc_gather_rows(word_emb, ids_flat, offs[c], CBLKS[c])
                   for c in range(len(CBLKS))]

    tri = (lax.broadcasted_iota(jnp.int32, (L, L), 0)
           <= lax.broadcasted_iota(jnp.int32, (L, L), 1)).astype(jnp.bfloat16)
    cidx = pl.pallas_call(
        _cidx_kernel,
        grid=(B // BBA,),
        in_specs=[
            pl.BlockSpec((BBA, L), lambda i: (i, 0)),
            pl.BlockSpec((BBA, L), lambda i: (i, 0)),
            pl.BlockSpec((L, L), lambda i: (0, 0)),
        ],
        out_specs=pl.BlockSpec((BBA, L), lambda i: (i, 0)),
        out_shape=jax.ShapeDtypeStruct((B, L), jnp.int32),
    )(ids32, segs32, tri)
    cidx3 = cidx.reshape(NBLK, 1, TOK)

    tab = (seg_emb[:, None, :]
           + pos_emb[None, :POS_BASE, :]).reshape(CTAB, E)
    tab = tab.astype(jnp.bfloat16)

    gamma2 = ln_gamma.reshape(1, E)
    beta2 = ln_beta.reshape(1, E)

    common_specs = [
        pl.BlockSpec((CTAB, E), lambda i: (0, 0)),
        pl.BlockSpec((1, E), lambda i: (0, 0)),
        pl.BlockSpec((1, E), lambda i: (0, 0)),
    ]
    out_shape = jax.ShapeDtypeStruct((NBLK, TOK, E), jnp.float32)

    out = None
    for c in range(len(CBLKS)):
        off, cblk = offs[c], CBLKS[c]
        w3 = wrow_chunks[c].reshape(cblk, TOK, E)
        in_specs = [
            pl.BlockSpec((1, TOK, E), lambda i: (i, 0, 0)),
            pl.BlockSpec((1, 1, TOK), lambda i, o=off: (i + o, 0, 0)),
        ] + common_specs
        args = [w3, cidx3, tab, gamma2, beta2]
        if c == 0:
            body, aliases = _finish_body, {}
        else:
            body, aliases = _finish_body_aliased, {5: 0}
            in_specs = in_specs + [pl.BlockSpec(memory_space=pl.ANY)]
            args.append(out)
        out = pl.pallas_call(
            body,
            grid=(cblk,),
            in_specs=in_specs,
            out_specs=pl.BlockSpec((1, TOK, E), lambda i, o=off: (i + o, 0, 0)),
            out_shape=out_shape,
            input_output_aliases=aliases,
        )(*args)
    return out.reshape(B, L, E)


# speedup vs baseline: 1.1626x; 1.1626x over previous
import functools

import jax
import jax.numpy as jnp
from jax import lax
from jax.experimental import pallas as pl
from jax.experimental.pallas import tpu as pltpu
from jax.experimental.pallas import tpu_sc as plsc

B, L, E = 1024, 200, 128
N = B * L
POS_BASE = 256
CTAB = 2 * POS_BASE
PAD_ID = 0
EPS = 1e-12
GATHER_W = 128
BBA = 256
BB = 32
TOK = BB * L
NBLK = B // BB
CBLKS = (4, 6, 10, 12)


def _sc_gather_rows(table, idx_flat, off_blocks, cblk):
    mesh = plsc.VectorSubcoreMesh(core_axis_name="c", subcore_axis_name="s")
    nc = cblk * TOK
    base = off_blocks * TOK // GATHER_W

    @functools.partial(
        pl.kernel,
        out_type=jax.ShapeDtypeStruct((nc, E), jnp.float32),
        mesh=mesh,
    )
    def gather_kernel(x_hbm, i_hbm, o_hbm):
        def body(i_vmem, o_vmem):
            pltpu.sync_copy(x_hbm.at[i_vmem.at[0]], o_vmem)

        pltpu.emit_pipeline(
            body,
            grid=(nc // GATHER_W,),
            in_specs=[pl.BlockSpec((1, GATHER_W), lambda i: (0, i + base))],
            out_specs=[pl.BlockSpec((GATHER_W, E), lambda i: (i, 0))],
            core_axis_name=("c", "s"),
            dimension_semantics=(pltpu.PARALLEL,),
        )(i_hbm, o_hbm)

    return gather_kernel(table, idx_flat)


def _cidx_kernel(ids_ref, seg_ref, tri_ref, cidx_ref):
    ids = ids_ref[...]
    mask = ids != PAD_ID
    mbf = mask.astype(jnp.bfloat16)
    posf = lax.dot_general(mbf, tri_ref[...], (((1,), (0,)), ((), ())),
                           preferred_element_type=jnp.float32)
    pos = jnp.where(mask, posf.astype(jnp.int32), 0)
    cidx_ref[...] = pos + POS_BASE * seg_ref[...]


def _finish_body(w_ref, cidx_ref, tab_ref, gamma_ref, beta_ref, out_ref):
    cid_t = jnp.transpose(cidx_ref[0])
    oh = (cid_t == lax.broadcasted_iota(jnp.int32, (1, CTAB), 1))
    oh = oh.astype(jnp.bfloat16)
    d23 = lax.dot_general(oh, tab_ref[...], (((1,), (0,)), ((), ())),
                          preferred_element_type=jnp.float32)

    t = w_ref[0] + d23
    mean = jnp.mean(t, axis=1, keepdims=True)
    tcen = t - mean
    var = jnp.mean(tcen * tcen, axis=1, keepdims=True)
    y = tcen * lax.rsqrt(var + EPS) * gamma_ref[...] + beta_ref[...]
    out_ref[0] = y


def _finish_body_aliased(w_ref, cidx_ref, tab_ref, gamma_ref, beta_ref,
                         prev_ref, out_ref):
    del prev_ref
    _finish_body(w_ref, cidx_ref, tab_ref, gamma_ref, beta_ref, out_ref)


def kernel(input_ids, segment_ids, word_emb, seg_emb, pos_emb, ln_gamma,
           ln_beta):
    ids32 = input_ids.astype(jnp.int32)
    segs32 = segment_ids.astype(jnp.int32)

    ids_flat = ids32.reshape(1, N)
    offs = [sum(CBLKS[:c]) for c in range(len(CBLKS))]
    wrow_chunks = [_sc_gather_rows(word_emb, ids_flat, offs[c], CBLKS[c])
                   for c in range(len(CBLKS))]

    tri = (lax.broadcasted_iota(jnp.int32, (L, L), 0)
           <= lax.broadcasted_iota(jnp.int32, (L, L), 1)).astype(jnp.bfloat16)
    cidx = pl.pallas_call(
        _cidx_kernel,
        grid=(B // BBA,),
        in_specs=[
            pl.BlockSpec((BBA, L), lambda i: (i, 0)),
            pl.BlockSpec((BBA, L), lambda i: (i, 0)),
            pl.BlockSpec((L, L), lambda i: (0, 0)),
        ],
        out_specs=pl.BlockSpec((BBA, L), lambda i: (i, 0)),
        out_shape=jax.ShapeDtypeStruct((B, L), jnp.int32),
    )(ids32, segs32, tri)
    cidx3 = cidx.reshape(NBLK, 1, TOK)

    tab = (seg_emb[:, None, :]
           + pos_emb[None, :POS_BASE, :]).reshape(CTAB, E)
    tab = tab.astype(jnp.bfloat16)

    gamma2 = ln_gamma.reshape(1, E)
    beta2 = ln_beta.reshape(1, E)

    common_specs = [
        pl.BlockSpec((CTAB, E), lambda i: (0, 0)),
        pl.BlockSpec((1, E), lambda i: (0, 0)),
        pl.BlockSpec((1, E), lambda i: (0, 0)),
    ]
    out_shape = jax.ShapeDtypeStruct((NBLK, TOK, E), jnp.float32)

    out = None
    for c in range(len(CBLKS)):
        off, cblk = offs[c], CBLKS[c]
        w3 = wrow_chunks[c].reshape(cblk, TOK, E)
        in_specs = [
            pl.BlockSpec((1, TOK, E), lambda i: (i, 0, 0)),
            pl.BlockSpec((1, 1, TOK), lambda i, o=off: (i + o, 0, 0)),
        ] + common_specs
        args = [w3, cidx3, tab, gamma2, beta2]
        if c == 0:
            body, aliases = _finish_body, {}
        else:
            body, aliases = _finish_body_aliased, {5: 0}
            in_specs = in_specs + [pl.BlockSpec(memory_space=pl.ANY)]
            args.append(out)
        out = pl.pallas_call(
            body,
            grid=(cblk,),
            in_specs=in_specs,
            out_specs=pl.BlockSpec((1, TOK, E), lambda i, o=off: (i + o, 0, 0)),
            out_shape=out_shape,
            input_output_aliases=aliases,
        )(*args)
    return out.reshape(B, L, E)
